# baseline (speedup 1.0000x reference)
import functools

import jax
import jax.numpy as jnp
from jax import lax
from jax.experimental import pallas as pl
from jax.experimental.pallas import tpu as pltpu
from jax.experimental.pallas import tpu_sc as plsc

N_NODES = 10000
N_EDGES = 320000
D = 128
E_DIM = 16

NW = 32
CH = 128
NCHT = N_EDGES // CH
SLICES = (875, 875, 750)
RPS = 624

_EPS = 1e-5


def _ln(h, gamma, beta):
    mu = jnp.mean(h, axis=-1, keepdims=True)
    var = jnp.mean((h - mu) ** 2, axis=-1, keepdims=True)
    return (h - mu) * lax.rsqrt(var + _EPS) * gamma + beta


def _worker_chunks(wid, total):
    q, r = total // NW, total % NW
    count = q + (wid < r).astype(jnp.int32)
    base = q * wid + jnp.minimum(wid, r)
    return base, count


def _npairs(total):
    q, r = total // NW, total % NW
    return (q + (1 if r else 0) + 2 + 1) // 2 + 1


def _premix_body(x_ref, w1s_ref, w1r_ref, be1_ref, a_ref, b_ref):
    x = x_ref[...]
    a_ref[...] = jnp.dot(x, w1s_ref[...], preferred_element_type=jnp.float32)
    b_ref[...] = jnp.dot(x, w1r_ref[...], preferred_element_type=jnp.float32) + be1_ref[...]


def _premix(x, w1s, w1r, b_e1):
    blk = 2000
    grid = N_NODES // blk
    return pl.pallas_call(
        _premix_body,
        grid=(grid,),
        in_specs=[
            pl.BlockSpec((blk, D), lambda i: (i, 0)),
            pl.BlockSpec((D, D), lambda i: (0, 0)),
            pl.BlockSpec((D, D), lambda i: (0, 0)),
            pl.BlockSpec((1, D), lambda i: (0, 0)),
        ],
        out_specs=[
            pl.BlockSpec((blk, D), lambda i: (i, 0)),
            pl.BlockSpec((blk, D), lambda i: (i, 0)),
        ],
        out_shape=[
            jax.ShapeDtypeStruct((N_NODES, D), jnp.float32),
            jax.ShapeDtypeStruct((N_NODES, D), jnp.float32),
        ],
    )(x, w1s, w1r, b_e1.reshape(1, D))


def _gather_body(chunk0, nchunks, send_ref, recv_ref, a_ref, b_ref, g_ref,
                 idx_s0, idx_s1, idx_r0, idx_r1,
                 bufa0, bufa1, bufb0, bufb1, g0, g1,
                 sem_i0, sem_i1, sem_g0, sem_g1, sem_o0, sem_o1):
    cid = lax.axis_index("c")
    sid = lax.axis_index("s")
    wid = sid * 2 + cid
    base, count = _worker_chunks(wid, nchunks)

    idx_s = (idx_s0, idx_s1)
    idx_r = (idx_r0, idx_r1)
    bufa = (bufa0, bufa1)
    bufb = (bufb0, bufb1)
    gbuf = (g0, g1)
    sem_i = (sem_i0, sem_i1)
    sem_g = (sem_g0, sem_g1)
    sem_o = (sem_o0, sem_o1)

    def idx_start(t, b):
        pltpu.async_copy(send_ref.at[chunk0 + base + t], idx_s[b], sem_i[b])
        pltpu.async_copy(recv_ref.at[chunk0 + base + t], idx_r[b], sem_i[b])

    def idx_wait(b):
        pltpu.make_async_copy(send_ref.at[0], idx_s[b], sem_i[b]).wait()
        pltpu.make_async_copy(recv_ref.at[0], idx_r[b], sem_i[b]).wait()

    def gat_start(b):
        pltpu.async_copy(a_ref.at[idx_s[b]], bufa[b], sem_g[b])
        pltpu.async_copy(b_ref.at[idx_r[b]], bufb[b], sem_g[b])

    def gat_wait(b):
        pltpu.make_async_copy(a_ref.at[idx_s[b]], bufa[b], sem_g[b]).wait()
        pltpu.make_async_copy(b_ref.at[idx_r[b]], bufb[b], sem_g[b]).wait()

    def wb_wait(b):
        pltpu.make_async_copy(gbuf[b], g_ref.at[pl.ds(0, CH)], sem_o[b]).wait()

    round_c = jnp.full((16,), 0x8000, jnp.uint32)
    mask_hi = jnp.full((16,), 0xFFFF0000, jnp.uint32)
    sh16 = jnp.full((16,), 16, jnp.uint32)

    def add_wb(t, b):
        def row_group(rr, carry):
            for j in range(2):
                r = rr * 2 + j
                for k in range(4):
                    sl = pl.ds(k * 16, 16)
                    sh = pl.ds(64 + k * 16, 16)
                    s_lo = bufa[b][r, sl] + bufb[b][r, sl]
                    s_hi = bufa[b][r, sh] + bufb[b][r, sh]
                    u_lo = jax.lax.bitcast_convert_type(s_lo, jnp.uint32)
                    u_hi = jax.lax.bitcast_convert_type(s_hi, jnp.uint32)
                    w = (((u_lo + round_c) >> sh16)
                         | ((u_hi + round_c) & mask_hi))
                    gbuf[b][r, sl] = jax.lax.bitcast_convert_type(w, jnp.float32)
            return carry

        lax.fori_loop(0, CH // 2, row_group, 0)
        pltpu.async_copy(gbuf[b], g_ref.at[pl.ds((base + t) * CH, CH)], sem_o[b])

    def pair(p, carry):
        for bb in range(2):
            t = p * 2 + bb
            b = bb

            @pl.when((t >= 2) & (t <= count + 1))
            def _wait_gather():
                gat_wait(b)

            @pl.when(t < count)
            def _prefetch_idx():
                idx_start(t, b)

            @pl.when((t >= 1) & (t <= count))
            def _start_gather():
                idx_wait(1 - b)
                gat_start(1 - b)

            @pl.when((t >= 2) & (t <= count + 1))
            def _add_writeback():
                @pl.when(t >= 4)
                def _wb_drain():
                    wb_wait(b)
                add_wb(t - 2, b)

        return carry

    lax.fori_loop(0, _npairs(nchunks), pair, 0)
    wb_wait(0)
    wb_wait(1)


def _edge_gather(chunk0, nchunks, send2d, recv2d, a, b):
    mesh = plsc.VectorSubcoreMesh(core_axis_name="c", subcore_axis_name="s")
    return pl.kernel(
        functools.partial(_gather_body, chunk0, nchunks),
        out_type=jax.ShapeDtypeStruct((nchunks * CH, D // 2), jnp.float32),
        mesh=mesh,
        scratch_types=[
            pltpu.VMEM((CH,), jnp.int32), pltpu.VMEM((CH,), jnp.int32),
            pltpu.VMEM((CH,), jnp.int32), pltpu.VMEM((CH,), jnp.int32),
            pltpu.VMEM((CH, D), jnp.float32), pltpu.VMEM((CH, D), jnp.float32),
            pltpu.VMEM((CH, D), jnp.float32), pltpu.VMEM((CH, D), jnp.float32),
            pltpu.VMEM((CH, D // 2), jnp.float32), pltpu.VMEM((CH, D // 2), jnp.float32),
            pltpu.SemaphoreType.DMA, pltpu.SemaphoreType.DMA,
            pltpu.SemaphoreType.DMA, pltpu.SemaphoreType.DMA,
            pltpu.SemaphoreType.DMA, pltpu.SemaphoreType.DMA,
        ],
    )(send2d, recv2d, a, b)


def _halves(p):
    u = jax.lax.bitcast_convert_type(p, jnp.uint32)
    lo = jax.lax.bitcast_convert_type(u << 16, jnp.float32)
    hi = jax.lax.bitcast_convert_type(u & jnp.uint32(0xFFFF0000), jnp.float32)
    return lo, hi


def _edge_mlp_body(g_ref, ea_ref, w1el_ref, w1eh_ref, w2l_ref,
                   w2h_ref, b2_ref, ge_ref, bee_ref, msg_ref):
    ea = ea_ref[...]
    g_lo, g_hi = _halves(g_ref[...])
    ec_l = jnp.dot(ea, w1el_ref[...], preferred_element_type=jnp.float32)
    ec_h = jnp.dot(ea, w1eh_ref[...], preferred_element_type=jnp.float32)
    h_l = jnp.maximum(g_lo + ec_l, 0.0)
    h_h = jnp.maximum(g_hi + ec_h, 0.0)
    m = jnp.dot(h_l, w2l_ref[...], preferred_element_type=jnp.float32)
    m = m + jnp.dot(h_h, w2h_ref[...], preferred_element_type=jnp.float32)
    m = jnp.maximum(m + b2_ref[...], 0.0)
    msg_ref[...] = _ln(m, ge_ref[...], bee_ref[...])


def _edge_mlp(g, ea, ea_blk0, w1e, w_e2, b_e2, g_e, be_e):
    blk = 1600
    nrows = g.shape[0]
    grid = nrows // blk
    return pl.pallas_call(
        _edge_mlp_body,
        grid=(grid,),
        in_specs=[
            pl.BlockSpec((blk, D // 2), lambda i: (i, 0)),
            pl.BlockSpec((blk, E_DIM), lambda i: (i + ea_blk0, 0)),
            pl.BlockSpec((E_DIM, D // 2), lambda i: (0, 0)),
            pl.BlockSpec((E_DIM, D // 2), lambda i: (0, 0)),
            pl.BlockSpec((D // 2, D), lambda i: (0, 0)),
            pl.BlockSpec((D // 2, D), lambda i: (0, 0)),
            pl.BlockSpec((1, D), lambda i: (0, 0)),
            pl.BlockSpec((1, D), lambda i: (0, 0)),
            pl.BlockSpec((1, D), lambda i: (0, 0)),
        ],
        out_specs=pl.BlockSpec((blk, D), lambda i: (i, 0)),
        out_shape=jax.ShapeDtypeStruct((nrows, D), jnp.float32),
    )(g, ea, w1e[:, :D // 2], w1e[:, D // 2:], w_e2[:D // 2],
      w_e2[D // 2:], b_e2.reshape(1, D), g_e.reshape(1, D),
      be_e.reshape(1, D))


def _scatter_body(chunk0, nchunks, recv_ref, msg_ref, init_ref, out_ref,
                  idx0, idx1, rows0, rows1, acc,
                  sem_f0, sem_f1, sem_s0, sem_s1):
    cid = lax.axis_index("c")
    sid = lax.axis_index("s")
    wid = sid * 2 + cid
    base, count = _worker_chunks(wid, nchunks)

    idx = (idx0, idx1)
    rows = (rows0, rows1)
    sem_f = (sem_f0, sem_f1)
    sem_s = (sem_s0, sem_s1)

    init_off = cid * (init_ref.shape[0] // 2) if init_ref.shape[0] == 2 * N_NODES else 0

    pltpu.sync_copy(init_ref.at[pl.ds(init_off + sid * RPS, RPS)],
                    acc.at[pl.ds(sid * RPS, RPS)])

    @pl.when(sid == 15)
    def _init_tail():
        pltpu.sync_copy(init_ref.at[pl.ds(init_off + 16 * RPS, N_NODES - 16 * RPS)],
                        acc.at[pl.ds(16 * RPS, N_NODES - 16 * RPS)])

    plsc.subcore_barrier()

    def fetch_start(t, b):
        pltpu.async_copy(recv_ref.at[chunk0 + base + t], idx[b], sem_f[b])
        pltpu.async_copy(msg_ref.at[pl.ds((base + t) * CH, CH)], rows[b], sem_f[b])

    def fetch_wait(b):
        pltpu.make_async_copy(recv_ref.at[0], idx[b], sem_f[b]).wait()
        pltpu.make_async_copy(msg_ref.at[pl.ds(0, CH)], rows[b], sem_f[b]).wait()

    def scat_start(b):
        pltpu.async_copy(rows[b], acc.at[idx[b]], sem_s[b], add=True)

    def scat_wait(b):
        pltpu.make_async_copy(rows[b], acc.at[idx[b]], sem_s[b]).wait()

    def pair(p, carry):
        for bb in range(2):
            t = p * 2 + bb
            b = bb

            @pl.when(t < count)
            def _fetch():
                @pl.when(t >= 2)
                def _scat_drain():
                    scat_wait(b)
                fetch_start(t, b)

            @pl.when((t >= 1) & (t <= count))
            def _scatter():
                fetch_wait(1 - b)
                scat_start(1 - b)

        return carry

    lax.fori_loop(0, _npairs(nchunks), pair, 0)
    scat_wait(0)
    scat_wait(1)

    plsc.subcore_barrier()
    pltpu.sync_copy(acc.at[pl.ds(sid * RPS, RPS)],
                    out_ref.at[pl.ds(cid * N_NODES + sid * RPS, RPS)])

    @pl.when(sid == 15)
    def _out_tail():
        pltpu.sync_copy(acc.at[pl.ds(16 * RPS, N_NODES - 16 * RPS)],
                        out_ref.at[pl.ds(cid * N_NODES + 16 * RPS,
                                         N_NODES - 16 * RPS)])


def _scatter_agg(chunk0, nchunks, recv2d, msg, init):
    mesh = plsc.VectorSubcoreMesh(core_axis_name="c", subcore_axis_name="s")
    return pl.kernel(
        functools.partial(_scatter_body, chunk0, nchunks),
        out_type=jax.ShapeDtypeStruct((2 * N_NODES, D), jnp.float32),
        mesh=mesh,
        scratch_types=[
            pltpu.VMEM((CH,), jnp.int32), pltpu.VMEM((CH,), jnp.int32),
            pltpu.VMEM((CH, D), jnp.float32), pltpu.VMEM((CH, D), jnp.float32),
            pltpu.VMEM_SHARED((N_NODES, D), jnp.float32),
            pltpu.SemaphoreType.DMA, pltpu.SemaphoreType.DMA,
            pltpu.SemaphoreType.DMA, pltpu.SemaphoreType.DMA,
        ],
    )(recv2d, msg, init)


def _node_mlp_body(x_ref, p0_ref, p1_ref, wn1a_ref, wn1b_ref, bn1_ref,
                   wn2_ref, bn2_ref, gn_ref, ben_ref, out_ref):
    x = x_ref[...]
    agg = p0_ref[...] + p1_ref[...]
    h = jnp.dot(x, wn1a_ref[...], preferred_element_type=jnp.float32)
    h = h + jnp.dot(agg, wn1b_ref[...], preferred_element_type=jnp.float32)
    h = jnp.maximum(h + bn1_ref[...], 0.0)
    upd = jnp.dot(h, wn2_ref[...], preferred_element_type=jnp.float32) + bn2_ref[...]
    out_ref[...] = _ln(x + upd, gn_ref[...], ben_ref[...])


def _node_mlp(x, partials, wn1a, wn1b, b_n1, w_n2, b_n2, g_n, be_n):
    blk = 2000
    grid = N_NODES // blk
    return pl.pallas_call(
        _node_mlp_body,
        grid=(grid,),
        in_specs=[
            pl.BlockSpec((blk, D), lambda i: (i, 0)),
            pl.BlockSpec((blk, D), lambda i: (i, 0)),
            pl.BlockSpec((blk, D), lambda i: (i + N_NODES // blk, 0)),
            pl.BlockSpec((D, D), lambda i: (0, 0)),
            pl.BlockSpec((D, D), lambda i: (0, 0)),
            pl.BlockSpec((1, D), lambda i: (0, 0)),
            pl.BlockSpec((D, D), lambda i: (0, 0)),
            pl.BlockSpec((1, D), lambda i: (0, 0)),
            pl.BlockSpec((1, D), lambda i: (0, 0)),
            pl.BlockSpec((1, D), lambda i: (0, 0)),
        ],
        out_specs=pl.BlockSpec((blk, D), lambda i: (i, 0)),
        out_shape=jax.ShapeDtypeStruct((N_NODES, D), jnp.float32),
    )(x, partials, partials, wn1a, wn1b, b_n1.reshape(1, D), w_n2,
      b_n2.reshape(1, D), g_n.reshape(1, D), be_n.reshape(1, D))


def kernel(x, edge_index, edge_attr, W_e1, b_e1, W_e2, b_e2, g_e, be_e,
           W_n1, b_n1, W_n2, b_n2, g_n, be_n):
    send2d = edge_index[0].astype(jnp.int32).reshape(NCHT, CH)
    recv2d = edge_index[1].astype(jnp.int32).reshape(NCHT, CH)

    w1s = W_e1[:D]
    w1r = W_e1[D:2 * D]
    w1e = W_e1[2 * D:]
    wn1a = W_n1[:D]
    wn1b = W_n1[D:]

    a, b = _premix(x, w1s, w1r, b_e1)

    offs = [sum(SLICES[:s]) for s in range(len(SLICES))]
    gathered = [_edge_gather(off, n, send2d, recv2d, a, b)
                for off, n in zip(offs, SLICES)]
    msgs = [_edge_mlp(g, edge_attr, off * CH // 1600, w1e, W_e2, b_e2, g_e, be_e)
            for (off, n), g in zip(zip(offs, SLICES), gathered)]

    partials = jnp.zeros((N_NODES, D), jnp.float32)
    for (off, n), m in zip(zip(offs, SLICES), msgs):
        partials = _scatter_agg(off, n, recv2d, m, partials)

    return _node_mlp(x, partials, wn1a, wn1b, b_n1, W_n2, b_n2, g_n, be_n)


# speedup vs baseline: 1.0293x; 1.0293x over previous
import functools

import jax
import jax.numpy as jnp
from jax import lax
from jax.experimental import pallas as pl
from jax.experimental.pallas import tpu as pltpu
from jax.experimental.pallas import tpu_sc as plsc

N_NODES = 10000
N_EDGES = 320000
D = 128
E_DIM = 16

NW = 32
CH = 128
NCHT = N_EDGES // CH
SLICES = (1250, 1250)
RPS = 624

_EPS = 1e-5


def _ln(h, gamma, beta):
    mu = jnp.mean(h, axis=-1, keepdims=True)
    var = jnp.mean((h - mu) ** 2, axis=-1, keepdims=True)
    return (h - mu) * lax.rsqrt(var + _EPS) * gamma + beta


def _worker_chunks(wid, total):
    q, r = total // NW, total % NW
    count = q + (wid < r).astype(jnp.int32)
    base = q * wid + jnp.minimum(wid, r)
    return base, count


def _npairs(total):
    q, r = total // NW, total % NW
    return (q + (1 if r else 0) + 2 + 1) // 2 + 1


def _premix_body(x_ref, w1s_ref, w1r_ref, be1_ref, a_ref, b_ref):
    x = x_ref[...]
    a_ref[...] = jnp.dot(x, w1s_ref[...], preferred_element_type=jnp.float32)
    b_ref[...] = jnp.dot(x, w1r_ref[...], preferred_element_type=jnp.float32) + be1_ref[...]


def _premix(x, w1s, w1r, b_e1):
    blk = 2000
    grid = N_NODES // blk
    return pl.pallas_call(
        _premix_body,
        grid=(grid,),
        in_specs=[
            pl.BlockSpec((blk, D), lambda i: (i, 0)),
            pl.BlockSpec((D, D), lambda i: (0, 0)),
            pl.BlockSpec((D, D), lambda i: (0, 0)),
            pl.BlockSpec((1, D), lambda i: (0, 0)),
        ],
        out_specs=[
            pl.BlockSpec((blk, D), lambda i: (i, 0)),
            pl.BlockSpec((blk, D), lambda i: (i, 0)),
        ],
        out_shape=[
            jax.ShapeDtypeStruct((N_NODES, D), jnp.float32),
            jax.ShapeDtypeStruct((N_NODES, D), jnp.float32),
        ],
    )(x, w1s, w1r, b_e1.reshape(1, D))


def _gather_body(chunk0, nchunks, send_ref, recv_ref, a_ref, b_ref, g_ref,
                 idx_s0, idx_s1, idx_r0, idx_r1,
                 bufa0, bufa1, bufb0, bufb1, g0, g1,
                 sem_i0, sem_i1, sem_g0, sem_g1, sem_o0, sem_o1):
    cid = lax.axis_index("c")
    sid = lax.axis_index("s")
    wid = sid * 2 + cid
    base, count = _worker_chunks(wid, nchunks)

    idx_s = (idx_s0, idx_s1)
    idx_r = (idx_r0, idx_r1)
    bufa = (bufa0, bufa1)
    bufb = (bufb0, bufb1)
    gbuf = (g0, g1)
    sem_i = (sem_i0, sem_i1)
    sem_g = (sem_g0, sem_g1)
    sem_o = (sem_o0, sem_o1)

    def idx_start(t, b):
        pltpu.async_copy(send_ref.at[chunk0 + base + t], idx_s[b], sem_i[b])
        pltpu.async_copy(recv_ref.at[chunk0 + base + t], idx_r[b], sem_i[b])

    def idx_wait(b):
        pltpu.make_async_copy(send_ref.at[0], idx_s[b], sem_i[b]).wait()
        pltpu.make_async_copy(recv_ref.at[0], idx_r[b], sem_i[b]).wait()

    def gat_start(b):
        pltpu.async_copy(a_ref.at[idx_s[b]], bufa[b], sem_g[b])
        pltpu.async_copy(b_ref.at[idx_r[b]], bufb[b], sem_g[b])

    def gat_wait(b):
        pltpu.make_async_copy(a_ref.at[idx_s[b]], bufa[b], sem_g[b]).wait()
        pltpu.make_async_copy(b_ref.at[idx_r[b]], bufb[b], sem_g[b]).wait()

    def wb_wait(b):
        pltpu.make_async_copy(gbuf[b], g_ref.at[pl.ds(0, CH)], sem_o[b]).wait()

    round_c = jnp.full((16,), 0x8000, jnp.uint32)
    mask_hi = jnp.full((16,), 0xFFFF0000, jnp.uint32)
    sh16 = jnp.full((16,), 16, jnp.uint32)

    def add_wb(t, b):
        def row_group(rr, carry):
            for j in range(2):
                r = rr * 2 + j
                for k in range(4):
                    sl = pl.ds(k * 16, 16)
                    sh = pl.ds(64 + k * 16, 16)
                    s_lo = bufa[b][r, sl] + bufb[b][r, sl]
                    s_hi = bufa[b][r, sh] + bufb[b][r, sh]
                    u_lo = jax.lax.bitcast_convert_type(s_lo, jnp.uint32)
                    u_hi = jax.lax.bitcast_convert_type(s_hi, jnp.uint32)
                    w = (((u_lo + round_c) >> sh16)
                         | ((u_hi + round_c) & mask_hi))
                    gbuf[b][r, sl] = jax.lax.bitcast_convert_type(w, jnp.float32)
            return carry

        lax.fori_loop(0, CH // 2, row_group, 0)
        pltpu.async_copy(gbuf[b], g_ref.at[pl.ds((base + t) * CH, CH)], sem_o[b])

    def pair(p, carry):
        for bb in range(2):
            t = p * 2 + bb
            b = bb

            @pl.when((t >= 2) & (t <= count + 1))
            def _wait_gather():
                gat_wait(b)

            @pl.when(t < count)
            def _prefetch_idx():
                idx_start(t, b)

            @pl.when((t >= 1) & (t <= count))
            def _start_gather():
                idx_wait(1 - b)
                gat_start(1 - b)

            @pl.when((t >= 2) & (t <= count + 1))
            def _add_writeback():
                @pl.when(t >= 4)
                def _wb_drain():
                    wb_wait(b)
                add_wb(t - 2, b)

        return carry

    lax.fori_loop(0, _npairs(nchunks), pair, 0)
    wb_wait(0)
    wb_wait(1)


def _edge_gather(chunk0, nchunks, send2d, recv2d, a, b):
    mesh = plsc.VectorSubcoreMesh(core_axis_name="c", subcore_axis_name="s")
    return pl.kernel(
        functools.partial(_gather_body, chunk0, nchunks),
        out_type=jax.ShapeDtypeStruct((nchunks * CH, D // 2), jnp.float32),
        mesh=mesh,
        scratch_types=[
            pltpu.VMEM((CH,), jnp.int32), pltpu.VMEM((CH,), jnp.int32),
            pltpu.VMEM((CH,), jnp.int32), pltpu.VMEM((CH,), jnp.int32),
            pltpu.VMEM((CH, D), jnp.float32), pltpu.VMEM((CH, D), jnp.float32),
            pltpu.VMEM((CH, D), jnp.float32), pltpu.VMEM((CH, D), jnp.float32),
            pltpu.VMEM((CH, D // 2), jnp.float32), pltpu.VMEM((CH, D // 2), jnp.float32),
            pltpu.SemaphoreType.DMA, pltpu.SemaphoreType.DMA,
            pltpu.SemaphoreType.DMA, pltpu.SemaphoreType.DMA,
            pltpu.SemaphoreType.DMA, pltpu.SemaphoreType.DMA,
        ],
    )(send2d, recv2d, a, b)


def _halves(p):
    u = jax.lax.bitcast_convert_type(p, jnp.uint32)
    lo = jax.lax.bitcast_convert_type(u << 16, jnp.float32)
    hi = jax.lax.bitcast_convert_type(u & jnp.uint32(0xFFFF0000), jnp.float32)
    return lo, hi


def _edge_mlp_body(g_ref, ea_ref, w1el_ref, w1eh_ref, w2l_ref,
                   w2h_ref, b2_ref, ge_ref, bee_ref, msg_ref):
    ea = ea_ref[...]
    g_lo, g_hi = _halves(g_ref[...])
    ec_l = jnp.dot(ea, w1el_ref[...], preferred_element_type=jnp.float32)
    ec_h = jnp.dot(ea, w1eh_ref[...], preferred_element_type=jnp.float32)
    h_l = jnp.maximum(g_lo + ec_l, 0.0)
    h_h = jnp.maximum(g_hi + ec_h, 0.0)
    m = jnp.dot(h_l, w2l_ref[...], preferred_element_type=jnp.float32)
    m = m + jnp.dot(h_h, w2h_ref[...], preferred_element_type=jnp.float32)
    m = jnp.maximum(m + b2_ref[...], 0.0)
    msg_ref[...] = _ln(m, ge_ref[...], bee_ref[...])


def _edge_mlp(g, ea, ea_blk0, w1e, w_e2, b_e2, g_e, be_e):
    blk = 2000
    nrows = g.shape[0]
    grid = nrows // blk
    return pl.pallas_call(
        _edge_mlp_body,
        grid=(grid,),
        in_specs=[
            pl.BlockSpec((blk, D // 2), lambda i: (i, 0)),
            pl.BlockSpec((blk, E_DIM), lambda i: (i + ea_blk0, 0)),
            pl.BlockSpec((E_DIM, D // 2), lambda i: (0, 0)),
            pl.BlockSpec((E_DIM, D // 2), lambda i: (0, 0)),
            pl.BlockSpec((D // 2, D), lambda i: (0, 0)),
            pl.BlockSpec((D // 2, D), lambda i: (0, 0)),
            pl.BlockSpec((1, D), lambda i: (0, 0)),
            pl.BlockSpec((1, D), lambda i: (0, 0)),
            pl.BlockSpec((1, D), lambda i: (0, 0)),
        ],
        out_specs=pl.BlockSpec((blk, D), lambda i: (i, 0)),
        out_shape=jax.ShapeDtypeStruct((nrows, D), jnp.float32),
    )(g, ea, w1e[:, :D // 2], w1e[:, D // 2:], w_e2[:D // 2],
      w_e2[D // 2:], b_e2.reshape(1, D), g_e.reshape(1, D),
      be_e.reshape(1, D))


def _scatter_body(chunk0, nchunks, recv_ref, msg_ref, init_ref, out_ref,
                  idx0, idx1, rows0, rows1, acc,
                  sem_f0, sem_f1, sem_s0, sem_s1):
    cid = lax.axis_index("c")
    sid = lax.axis_index("s")
    wid = sid * 2 + cid
    base, count = _worker_chunks(wid, nchunks)

    idx = (idx0, idx1)
    rows = (rows0, rows1)
    sem_f = (sem_f0, sem_f1)
    sem_s = (sem_s0, sem_s1)

    init_off = cid * (init_ref.shape[0] // 2) if init_ref.shape[0] == 2 * N_NODES else 0

    pltpu.sync_copy(init_ref.at[pl.ds(init_off + sid * RPS, RPS)],
                    acc.at[pl.ds(sid * RPS, RPS)])

    @pl.when(sid == 15)
    def _init_tail():
        pltpu.sync_copy(init_ref.at[pl.ds(init_off + 16 * RPS, N_NODES - 16 * RPS)],
                        acc.at[pl.ds(16 * RPS, N_NODES - 16 * RPS)])

    plsc.subcore_barrier()

    def fetch_start(t, b):
        pltpu.async_copy(recv_ref.at[chunk0 + base + t], idx[b], sem_f[b])
        pltpu.async_copy(msg_ref.at[pl.ds((base + t) * CH, CH)], rows[b], sem_f[b])

    def fetch_wait(b):
        pltpu.make_async_copy(recv_ref.at[0], idx[b], sem_f[b]).wait()
        pltpu.make_async_copy(msg_ref.at[pl.ds(0, CH)], rows[b], sem_f[b]).wait()

    def scat_start(b):
        pltpu.async_copy(rows[b], acc.at[idx[b]], sem_s[b], add=True)

    def scat_wait(b):
        pltpu.make_async_copy(rows[b], acc.at[idx[b]], sem_s[b]).wait()

    def pair(p, carry):
        for bb in range(2):
            t = p * 2 + bb
            b = bb

            @pl.when(t < count)
            def _fetch():
                @pl.when(t >= 2)
                def _scat_drain():
                    scat_wait(b)
                fetch_start(t, b)

            @pl.when((t >= 1) & (t <= count))
            def _scatter():
                fetch_wait(1 - b)
                scat_start(1 - b)

        return carry

    lax.fori_loop(0, _npairs(nchunks), pair, 0)
    scat_wait(0)
    scat_wait(1)

    plsc.subcore_barrier()
    pltpu.sync_copy(acc.at[pl.ds(sid * RPS, RPS)],
                    out_ref.at[pl.ds(cid * N_NODES + sid * RPS, RPS)])

    @pl.when(sid == 15)
    def _out_tail():
        pltpu.sync_copy(acc.at[pl.ds(16 * RPS, N_NODES - 16 * RPS)],
                        out_ref.at[pl.ds(cid * N_NODES + 16 * RPS,
                                         N_NODES - 16 * RPS)])


def _scatter_agg(chunk0, nchunks, recv2d, msg, init):
    mesh = plsc.VectorSubcoreMesh(core_axis_name="c", subcore_axis_name="s")
    return pl.kernel(
        functools.partial(_scatter_body, chunk0, nchunks),
        out_type=jax.ShapeDtypeStruct((2 * N_NODES, D), jnp.float32),
        mesh=mesh,
        scratch_types=[
            pltpu.VMEM((CH,), jnp.int32), pltpu.VMEM((CH,), jnp.int32),
            pltpu.VMEM((CH, D), jnp.float32), pltpu.VMEM((CH, D), jnp.float32),
            pltpu.VMEM_SHARED((N_NODES, D), jnp.float32),
            pltpu.SemaphoreType.DMA, pltpu.SemaphoreType.DMA,
            pltpu.SemaphoreType.DMA, pltpu.SemaphoreType.DMA,
        ],
    )(recv2d, msg, init)


def _node_mlp_body(x_ref, p0_ref, p1_ref, wn1a_ref, wn1b_ref, bn1_ref,
                   wn2_ref, bn2_ref, gn_ref, ben_ref, out_ref):
    x = x_ref[...]
    agg = p0_ref[...] + p1_ref[...]
    h = jnp.dot(x, wn1a_ref[...], preferred_element_type=jnp.float32)
    h = h + jnp.dot(agg, wn1b_ref[...], preferred_element_type=jnp.float32)
    h = jnp.maximum(h + bn1_ref[...], 0.0)
    upd = jnp.dot(h, wn2_ref[...], preferred_element_type=jnp.float32) + bn2_ref[...]
    out_ref[...] = _ln(x + upd, gn_ref[...], ben_ref[...])


def _node_mlp(x, partials, wn1a, wn1b, b_n1, w_n2, b_n2, g_n, be_n):
    blk = 2000
    grid = N_NODES // blk
    return pl.pallas_call(
        _node_mlp_body,
        grid=(grid,),
        in_specs=[
            pl.BlockSpec((blk, D), lambda i: (i, 0)),
            pl.BlockSpec((blk, D), lambda i: (i, 0)),
            pl.BlockSpec((blk, D), lambda i: (i + N_NODES // blk, 0)),
            pl.BlockSpec((D, D), lambda i: (0, 0)),
            pl.BlockSpec((D, D), lambda i: (0, 0)),
            pl.BlockSpec((1, D), lambda i: (0, 0)),
            pl.BlockSpec((D, D), lambda i: (0, 0)),
            pl.BlockSpec((1, D), lambda i: (0, 0)),
            pl.BlockSpec((1, D), lambda i: (0, 0)),
            pl.BlockSpec((1, D), lambda i: (0, 0)),
        ],
        out_specs=pl.BlockSpec((blk, D), lambda i: (i, 0)),
        out_shape=jax.ShapeDtypeStruct((N_NODES, D), jnp.float32),
    )(x, partials, partials, wn1a, wn1b, b_n1.reshape(1, D), w_n2,
      b_n2.reshape(1, D), g_n.reshape(1, D), be_n.reshape(1, D))


def kernel(x, edge_index, edge_attr, W_e1, b_e1, W_e2, b_e2, g_e, be_e,
           W_n1, b_n1, W_n2, b_n2, g_n, be_n):
    send2d = edge_index[0].astype(jnp.int32).reshape(NCHT, CH)
    recv2d = edge_index[1].astype(jnp.int32).reshape(NCHT, CH)

    w1s = W_e1[:D]
    w1r = W_e1[D:2 * D]
    w1e = W_e1[2 * D:]
    wn1a = W_n1[:D]
    wn1b = W_n1[D:]

    a, b = _premix(x, w1s, w1r, b_e1)

    offs = [sum(SLICES[:s]) for s in range(len(SLICES))]
    gathered = [_edge_gather(off, n, send2d, recv2d, a, b)
                for off, n in zip(offs, SLICES)]
    msgs = [_edge_mlp(g, edge_attr, off * CH // 2000, w1e, W_e2, b_e2, g_e, be_e)
            for (off, n), g in zip(zip(offs, SLICES), gathered)]

    partials = jnp.zeros((N_NODES, D), jnp.float32)
    for (off, n), m in zip(zip(offs, SLICES), msgs):
        partials = _scatter_agg(off, n, recv2d, m, partials)

    return _node_mlp(x, partials, wn1a, wn1b, b_n1, W_n2, b_n2, g_n, be_n)
